# Initial kernel scaffold; baseline (speedup 1.0000x reference)
import jax
import jax.numpy as jnp
from jax.experimental import pallas as pl


def kernel(x, edge_index, We1, be1, We2, be2, We3, be3, Wmu, bmu, Wlv, blv, Wd1, bd1, Wd2, bd2, Wd3, bd3, Wdm, bdm, Wds, bds):
    raise NotImplementedError("write your pallas kernel here")


# speedup vs baseline: 3.2757x; 3.2757x over previous
import functools

import jax
import jax.numpy as jnp
from jax import lax
from jax.experimental import pallas as pl
from jax.experimental.pallas import tpu as pltpu
from jax.experimental.pallas import tpu_sc as plsc

N = 10000
F = 256
E = 160000
NH = N // 2
PADH = 120
NHP = NH + PADH
NP = 2 * NHP
ZROWS = NP // 32
AROWS = ZROWS + 8
CH = 128
ALIGN = 16 * CH
EP = E + 2 * ALIGN
BR = 1024


def _sc_mesh():
    return plsc.VectorSubcoreMesh(core_axis_name="c", subcore_axis_name="s")


def _spmm(h, src_pad, dst_pad, meta, zeros, f):
    @functools.partial(
        pl.kernel,
        out_type=jax.ShapeDtypeStruct((32, ZROWS * f), jnp.float32),
        mesh=_sc_mesh(),
        scratch_types=[
            pltpu.VMEM((CH,), jnp.int32),
            pltpu.VMEM((CH,), jnp.int32),
            pltpu.VMEM((2, 16), jnp.int32),
            pltpu.VMEM((CH, f), jnp.float32),
            pltpu.VMEM((AROWS * f,), jnp.float32),
            pltpu.SemaphoreType.DMA,
        ],
    )
    def k(src_hbm, dst_hbm, meta_hbm, z_hbm, h_hbm, out_hbm,
          sidx_v, didx_v, meta_v, rows_v, acc_v, sem):
        c = lax.axis_index("c")
        s = lax.axis_index("s")
        w = c * 16 + s
        pltpu.sync_copy(z_hbm, acc_v)
        pltpu.sync_copy(meta_hbm.at[w], meta_v)
        mrow0 = meta_v[0]
        mrow1 = meta_v[1]
        start = mrow0[0]
        nch = mrow1[0]

        def chunk(i, carry):
            e = (start + i) * CH
            pltpu.sync_copy(src_hbm.at[pl.ds(e, CH)], sidx_v)
            pltpu.sync_copy(dst_hbm.at[pl.ds(e, CH)], didx_v)
            pltpu.async_copy(h_hbm.at[sidx_v], rows_v, sem).wait()

            def group(g, carry2):
                dvec = didx_v[pl.ds(g * 16, 16)]
                for ll in range(16):
                    off = dvec[ll] * f
                    gl = g * 16 + ll
                    for jj in range(f // 16):
                        vals = rows_v[gl, pl.ds(jj * 16, 16)]
                        plsc.addupdate(acc_v.at[pl.ds(off + jj * 16, 16)], vals)
                return carry2

            lax.fori_loop(0, CH // 16, group, 0)
            return carry

        lax.fori_loop(0, nch, chunk, 0)
        pltpu.sync_copy(acc_v.at[pl.ds(0, ZROWS * f)], out_hbm.at[w])

    return k(src_pad, dst_pad, meta, zeros, h).reshape(NP, f)


def _degree_counts(dst_pad, meta, zeros16):
    @functools.partial(
        pl.kernel,
        out_type=jax.ShapeDtypeStruct((32, ZROWS * 16), jnp.float32),
        mesh=_sc_mesh(),
        scratch_types=[
            pltpu.VMEM((CH,), jnp.int32),
            pltpu.VMEM((2, 16), jnp.int32),
            pltpu.VMEM((AROWS * 16,), jnp.float32),
            pltpu.SemaphoreType.DMA,
        ],
    )
    def k(dst_hbm, meta_hbm, z_hbm, out_hbm, didx_v, meta_v, acc_v, sem):
        ones16 = jnp.full((16,), 1.0, jnp.float32)
        c = lax.axis_index("c")
        s = lax.axis_index("s")
        w = c * 16 + s
        pltpu.sync_copy(z_hbm, acc_v)
        pltpu.sync_copy(meta_hbm.at[w], meta_v)
        mrow0 = meta_v[0]
        mrow1 = meta_v[1]
        start = mrow0[0]
        nch = mrow1[0]

        def chunk(i, carry):
            e = (start + i) * CH
            pltpu.sync_copy(dst_hbm.at[pl.ds(e, CH)], didx_v)

            def group(g, carry2):
                dvec = didx_v[pl.ds(g * 16, 16)]
                for ll in range(16):
                    off = dvec[ll] * 16
                    plsc.addupdate(acc_v.at[pl.ds(off, 16)], ones16)
                return carry2

            lax.fori_loop(0, CH // 16, group, 0)
            return carry

        lax.fori_loop(0, nch, chunk, 0)
        pltpu.sync_copy(acc_v.at[pl.ds(0, ZROWS * 16)], out_hbm.at[w])

    out = k(dst_pad, meta, zeros16).reshape(NP, 16)
    return out[:, :1]


def _row_spec(fw):
    return pl.BlockSpec((BR, fw), lambda i: (i, 0))


def _full_spec(r, c):
    return pl.BlockSpec((r, c), lambda i: (0, 0))


def _mm_pre(x, W, cnt):
    fi, fo = W.shape

    def body(x_ref, w_ref, c_ref, o_ref):
        dinv = lax.rsqrt(c_ref[...] + 1.0)
        o_ref[...] = jnp.dot(x_ref[...], w_ref[...],
                             preferred_element_type=jnp.float32) * dinv

    return pl.pallas_call(
        body, grid=(NP // BR,),
        in_specs=[_row_spec(fi), _full_spec(fi, fo), _row_spec(1)],
        out_specs=_row_spec(fo),
        out_shape=jax.ShapeDtypeStruct((NP, fo), jnp.float32),
    )(x, W, cnt)


def _mm_mid(S, hs, b, cnt, W):
    fi, fo = W.shape

    def body(s_ref, h_ref, b_ref, c_ref, w_ref, o_ref):
        dinv = lax.rsqrt(c_ref[...] + 1.0)
        y = jnp.maximum((s_ref[...] + h_ref[...]) * dinv + b_ref[...], 0.0)
        o_ref[...] = jnp.dot(y, w_ref[...],
                             preferred_element_type=jnp.float32) * dinv

    return pl.pallas_call(
        body, grid=(NP // BR,),
        in_specs=[_row_spec(fi), _row_spec(fi), _full_spec(1, fi),
                  _row_spec(1), _full_spec(fi, fo)],
        out_specs=_row_spec(fo),
        out_shape=jax.ShapeDtypeStruct((NP, fo), jnp.float32),
    )(S, hs, b.reshape(1, fi), cnt, W)


def _mm_z(S, hs, b, cnt, Wmu, bmu, Wlv, blv, eps, Wd1):
    fi = Wmu.shape[0]
    fo = Wd1.shape[1]

    def body(s_ref, h_ref, b_ref, c_ref, wmu_ref, bmu_ref, wlv_ref, blv_ref,
             e_ref, wd_ref, o_ref):
        dinv = lax.rsqrt(c_ref[...] + 1.0)
        henc = (s_ref[...] + h_ref[...]) * dinv + b_ref[...]
        mu = jnp.dot(henc, wmu_ref[...],
                     preferred_element_type=jnp.float32) + bmu_ref[...]
        lv = jnp.clip(jnp.dot(henc, wlv_ref[...],
                              preferred_element_type=jnp.float32)
                      + blv_ref[...], -10.0, 10.0)
        z = mu + jnp.exp(0.5 * lv) * e_ref[...]
        o_ref[...] = jnp.dot(z, wd_ref[...],
                             preferred_element_type=jnp.float32) * dinv

    return pl.pallas_call(
        body, grid=(NP // BR,),
        in_specs=[_row_spec(fi), _row_spec(fi), _full_spec(1, fi),
                  _row_spec(1), _full_spec(fi, fi), _full_spec(1, fi),
                  _full_spec(fi, fi), _full_spec(1, fi),
                  _row_spec(fi), _full_spec(fi, fo)],
        out_specs=_row_spec(fo),
        out_shape=jax.ShapeDtypeStruct((NP, fo), jnp.float32),
    )(S, hs, b.reshape(1, fi), cnt, Wmu, bmu.reshape(1, fi),
      Wlv, blv.reshape(1, fi), eps, Wd1)


def _mm_final(S, hs, b, cnt, Wdm, bdm, Wds, bds, eps2):
    fi = Wdm.shape[0]

    def body(s_ref, h_ref, b_ref, c_ref, wm_ref, bm_ref, ws_ref, bs_ref,
             e_ref, o_ref):
        dinv = lax.rsqrt(c_ref[...] + 1.0)
        d = (s_ref[...] + h_ref[...]) * dinv + b_ref[...]
        lmu = jnp.dot(d, wm_ref[...],
                      preferred_element_type=jnp.float32) + bm_ref[...]
        lls = jnp.clip(jnp.dot(d, ws_ref[...],
                               preferred_element_type=jnp.float32)
                       + bs_ref[...], -10.0, 3.0)
        o_ref[...] = jnp.exp(jnp.clip(lmu + jnp.exp(lls) * e_ref[...],
                                      -20.0, 20.0))

    return pl.pallas_call(
        body, grid=(NP // BR,),
        in_specs=[_row_spec(fi), _row_spec(fi), _full_spec(1, fi),
                  _row_spec(1), _full_spec(fi, fi), _full_spec(1, fi),
                  _full_spec(fi, fi), _full_spec(1, fi), _row_spec(fi)],
        out_specs=_row_spec(fi),
        out_shape=jax.ShapeDtypeStruct((NP, fi), jnp.float32),
    )(S, hs, b.reshape(1, fi), cnt, Wdm, bdm.reshape(1, fi),
      Wds, bds.reshape(1, fi), eps2)


def _pad_rows(a):
    f = a.shape[1]
    out = jnp.zeros((NP, f), a.dtype)
    out = lax.dynamic_update_slice(out, a[:NH], (0, 0))
    out = lax.dynamic_update_slice(out, a[NH:], (NHP, 0))
    return out


def _pad_w(W, fi, fo):
    return jnp.pad(W, ((0, fi - W.shape[0]), (0, fo - W.shape[1])))


def _pad_b(b, fo):
    return jnp.pad(b, (0, fo - b.shape[0]))


def _build_edges(edge_index):
    src = edge_index[0]
    dst = edge_index[1]
    srcrow0 = src + PADH * (src >= NH).astype(jnp.int32)
    dstrow0 = dst + PADH * (dst >= NH).astype(jnp.int32)
    key = dstrow0 // ZROWS
    order = jnp.argsort(key, stable=True)
    srcrow = srcrow0[order]
    dstrow = dstrow0[order]

    cnt_w = jnp.bincount(key, length=32).astype(jnp.int32)
    P_w = ((cnt_w + CH - 1) // CH) * CH
    CS = jnp.concatenate([jnp.zeros((1,), jnp.int32),
                          jnp.cumsum(cnt_w)[:-1].astype(jnp.int32)])
    S_w = jnp.concatenate([jnp.zeros((1,), jnp.int32),
                           jnp.cumsum(P_w)[:-1].astype(jnp.int32)])

    p = jnp.arange(EP, dtype=jnp.int32)
    w_of_p = jnp.clip(jnp.searchsorted(S_w, p, side="right") - 1, 0, 31)
    off = p - S_w[w_of_p]
    valid = off < cnt_w[w_of_p]
    jidx = jnp.clip(CS[w_of_p] + off, 0, E - 1)
    src_pad = jnp.where(valid, srcrow[jidx], NH)
    dst_pad = jnp.where(valid, dstrow[jidx] - w_of_p * ZROWS, ZROWS)

    meta = jnp.stack([
        jnp.broadcast_to((S_w // CH)[:, None], (32, 16)),
        jnp.broadcast_to((P_w // CH)[:, None], (32, 16)),
    ], axis=1).astype(jnp.int32)
    return src_pad, dst_pad, meta


def kernel(x, edge_index, We1, be1, We2, be2, We3, be3, Wmu, bmu, Wlv, blv,
           Wd1, bd1, Wd2, bd2, Wd3, bd3, Wdm, bdm, Wds, bds):
    src_pad, dst_pad, meta = _build_edges(edge_index)

    zeros = jnp.zeros((AROWS * F,), jnp.float32)

    xp = _pad_rows(x)
    eps = _pad_rows(jax.random.normal(jax.random.key(42), (N, 128), jnp.float32))
    eps2 = _pad_rows(jax.random.normal(jax.random.key(43), (N, F), jnp.float32))

    f1, f2, f3 = 256, 256, 128
    We1p = _pad_w(We1, F, f1)
    We2p = _pad_w(We2, f1, f2)
    We3p = _pad_w(We3, f2, f3)
    Wd1p = _pad_w(Wd1, f3, f2)
    Wd2p = _pad_w(Wd2, f2, f1)
    Wd3p = _pad_w(Wd3, f1, F)
    be1p = _pad_b(be1, f1)
    be2p = _pad_b(be2, f2)
    bd1p = _pad_b(bd1, f2)
    bd2p = _pad_b(bd2, f1)

    cnt = _degree_counts(dst_pad, meta, zeros[:AROWS * 16])

    hs1 = _mm_pre(xp, We1p, cnt)
    S1 = _spmm(hs1, src_pad, dst_pad, meta, zeros[:AROWS * f1], f1)
    hs2 = _mm_mid(S1, hs1, be1p, cnt, We2p)
    S2 = _spmm(hs2, src_pad, dst_pad, meta, zeros[:AROWS * f2], f2)
    hs3 = _mm_mid(S2, hs2, be2p, cnt, We3p)
    S3 = _spmm(hs3, src_pad, dst_pad, meta, zeros[:AROWS * f3], f3)
    hs4 = _mm_z(S3, hs3, be3, cnt, Wmu, bmu, Wlv, blv, eps, Wd1p)
    S4 = _spmm(hs4, src_pad, dst_pad, meta, zeros[:AROWS * f2], f2)
    hs5 = _mm_mid(S4, hs4, bd1p, cnt, Wd2p)
    S5 = _spmm(hs5, src_pad, dst_pad, meta, zeros[:AROWS * f1], f1)
    hs6 = _mm_mid(S5, hs5, bd2p, cnt, Wd3p)
    S6 = _spmm(hs6, src_pad, dst_pad, meta, zeros[:AROWS * F], F)
    outp = _mm_final(S6, hs6, bd3, cnt, Wdm, bdm, Wds, bds, eps2)

    return jnp.concatenate([outp[:NH], outp[NHP:NHP + NH]], axis=0)


# speedup vs baseline: 4.2082x; 1.2847x over previous
import functools

import jax
import jax.numpy as jnp
from jax import lax
from jax.experimental import pallas as pl
from jax.experimental.pallas import tpu as pltpu
from jax.experimental.pallas import tpu_sc as plsc

N = 10000
F = 256
E = 160000
NH = N // 2
PADH = 120
NHP = NH + PADH
NP = 2 * NHP
ZROWS = NP // 32
AROWS = ZROWS + 8
CH = 64
SB = 32
EP = E + 32 * CH + SB * CH
BR = 1024


def _sc_mesh():
    return plsc.VectorSubcoreMesh(core_axis_name="c", subcore_axis_name="s")


def _spmm(h, src_pad, dst_pad, meta, zeros, f):
    @functools.partial(
        pl.kernel,
        out_type=jax.ShapeDtypeStruct((32, ZROWS * f), jnp.float32),
        mesh=_sc_mesh(),
        scratch_types=[
            pltpu.VMEM((SB * CH,), jnp.int32),
            pltpu.VMEM((SB * CH,), jnp.int32),
            pltpu.VMEM((2, 16), jnp.int32),
            pltpu.VMEM((2 * CH, f), jnp.float32),
            pltpu.VMEM((AROWS * f,), jnp.float32),
            pltpu.SemaphoreType.DMA,
        ],
    )
    def k(src_hbm, dst_hbm, meta_hbm, z_hbm, h_hbm, out_hbm,
          sidx_v, didx_v, meta_v, rows_v, acc_v, sem):
        c = lax.axis_index("c")
        s = lax.axis_index("s")
        w = c * 16 + s
        pltpu.sync_copy(z_hbm, acc_v)
        pltpu.sync_copy(meta_hbm.at[w], meta_v)
        mrow0 = meta_v[0]
        mrow1 = meta_v[1]
        start = mrow0[0]
        nch = mrow1[0]

        def superchunk(sb, carry):
            base_ch = sb * SB
            e0 = (start + base_ch) * CH
            nin = jnp.minimum(SB, nch - base_ch)
            pltpu.sync_copy(src_hbm.at[pl.ds(e0, SB * CH)], sidx_v)
            pltpu.sync_copy(dst_hbm.at[pl.ds(e0, SB * CH)], didx_v)
            pltpu.async_copy(h_hbm.at[sidx_v.at[pl.ds(0, CH)]],
                             rows_v.at[pl.ds(0, CH)], sem)

            def chunk(cc, carry2):
                b = lax.rem(cc, 2)
                nb = lax.rem(cc + 1, 2)
                pltpu.make_async_copy(h_hbm.at[pl.ds(0, CH)],
                                      rows_v.at[pl.ds(0, CH)], sem).wait()

                @pl.when(cc + 1 < nin)
                def _():
                    pltpu.async_copy(
                        h_hbm.at[sidx_v.at[pl.ds((cc + 1) * CH, CH)]],
                        rows_v.at[pl.ds(nb * CH, CH)], sem)

                rowbase = b * CH

                def group(g, carry3):
                    dvec = didx_v[pl.ds(cc * CH + g * 16, 16)]
                    for ll in range(16):
                        off = dvec[ll] * f
                        gl = rowbase + g * 16 + ll
                        for jj in range(f // 16):
                            vals = rows_v[gl, pl.ds(jj * 16, 16)]
                            plsc.addupdate(acc_v.at[pl.ds(off + jj * 16, 16)],
                                           vals)
                    return carry3

                lax.fori_loop(0, CH // 16, group, 0)
                return carry2

            lax.fori_loop(0, nin, chunk, 0)
            return carry

        nsb = (nch + SB - 1) // SB
        lax.fori_loop(0, nsb, superchunk, 0)
        pltpu.sync_copy(acc_v.at[pl.ds(0, ZROWS * f)], out_hbm.at[w])

    return k(src_pad, dst_pad, meta, zeros, h).reshape(NP, f)


def _degree_counts(dst_pad, meta, zeros16):
    @functools.partial(
        pl.kernel,
        out_type=jax.ShapeDtypeStruct((32, ZROWS * 16), jnp.float32),
        mesh=_sc_mesh(),
        scratch_types=[
            pltpu.VMEM((SB * CH,), jnp.int32),
            pltpu.VMEM((2, 16), jnp.int32),
            pltpu.VMEM((AROWS * 16,), jnp.float32),
            pltpu.SemaphoreType.DMA,
        ],
    )
    def k(dst_hbm, meta_hbm, z_hbm, out_hbm, didx_v, meta_v, acc_v, sem):
        ones16 = jnp.full((16,), 1.0, jnp.float32)
        c = lax.axis_index("c")
        s = lax.axis_index("s")
        w = c * 16 + s
        pltpu.sync_copy(z_hbm, acc_v)
        pltpu.sync_copy(meta_hbm.at[w], meta_v)
        mrow0 = meta_v[0]
        mrow1 = meta_v[1]
        start = mrow0[0]
        nch = mrow1[0]

        def superchunk(sb, carry):
            base_ch = sb * SB
            nin = jnp.minimum(SB, nch - base_ch)
            pltpu.sync_copy(dst_hbm.at[pl.ds((start + base_ch) * CH, SB * CH)],
                            didx_v)

            def group(g, carry2):
                dvec = didx_v[pl.ds(g * 16, 16)]
                for ll in range(16):
                    off = dvec[ll] * 16
                    plsc.addupdate(acc_v.at[pl.ds(off, 16)], ones16)
                return carry2

            lax.fori_loop(0, nin * (CH // 16), group, 0)
            return carry

        nsb = (nch + SB - 1) // SB
        lax.fori_loop(0, nsb, superchunk, 0)
        pltpu.sync_copy(acc_v.at[pl.ds(0, ZROWS * 16)], out_hbm.at[w])

    out = k(dst_pad, meta, zeros16).reshape(NP, 16)
    return out[:, :1]


def _row_spec(fw):
    return pl.BlockSpec((BR, fw), lambda i: (i, 0))


def _full_spec(r, c):
    return pl.BlockSpec((r, c), lambda i: (0, 0))


def _mm_pre(x, W, cnt):
    fi, fo = W.shape

    def body(x_ref, w_ref, c_ref, o_ref):
        dinv = lax.rsqrt(c_ref[...] + 1.0)
        o_ref[...] = jnp.dot(x_ref[...], w_ref[...],
                             preferred_element_type=jnp.float32) * dinv

    return pl.pallas_call(
        body, grid=(NP // BR,),
        in_specs=[_row_spec(fi), _full_spec(fi, fo), _row_spec(1)],
        out_specs=_row_spec(fo),
        out_shape=jax.ShapeDtypeStruct((NP, fo), jnp.float32),
    )(x, W, cnt)


def _mm_mid(S, hs, b, cnt, W):
    fi, fo = W.shape

    def body(s_ref, h_ref, b_ref, c_ref, w_ref, o_ref):
        dinv = lax.rsqrt(c_ref[...] + 1.0)
        y = jnp.maximum((s_ref[...] + h_ref[...]) * dinv + b_ref[...], 0.0)
        o_ref[...] = jnp.dot(y, w_ref[...],
                             preferred_element_type=jnp.float32) * dinv

    return pl.pallas_call(
        body, grid=(NP // BR,),
        in_specs=[_row_spec(fi), _row_spec(fi), _full_spec(1, fi),
                  _row_spec(1), _full_spec(fi, fo)],
        out_specs=_row_spec(fo),
        out_shape=jax.ShapeDtypeStruct((NP, fo), jnp.float32),
    )(S, hs, b.reshape(1, fi), cnt, W)


def _mm_z(S, hs, b, cnt, Wmu, bmu, Wlv, blv, eps, Wd1):
    fi = Wmu.shape[0]
    fo = Wd1.shape[1]

    def body(s_ref, h_ref, b_ref, c_ref, wmu_ref, bmu_ref, wlv_ref, blv_ref,
             e_ref, wd_ref, o_ref):
        dinv = lax.rsqrt(c_ref[...] + 1.0)
        henc = (s_ref[...] + h_ref[...]) * dinv + b_ref[...]
        mu = jnp.dot(henc, wmu_ref[...],
                     preferred_element_type=jnp.float32) + bmu_ref[...]
        lv = jnp.clip(jnp.dot(henc, wlv_ref[...],
                              preferred_element_type=jnp.float32)
                      + blv_ref[...], -10.0, 10.0)
        z = mu + jnp.exp(0.5 * lv) * e_ref[...]
        o_ref[...] = jnp.dot(z, wd_ref[...],
                             preferred_element_type=jnp.float32) * dinv

    return pl.pallas_call(
        body, grid=(NP // BR,),
        in_specs=[_row_spec(fi), _row_spec(fi), _full_spec(1, fi),
                  _row_spec(1), _full_spec(fi, fi), _full_spec(1, fi),
                  _full_spec(fi, fi), _full_spec(1, fi),
                  _row_spec(fi), _full_spec(fi, fo)],
        out_specs=_row_spec(fo),
        out_shape=jax.ShapeDtypeStruct((NP, fo), jnp.float32),
    )(S, hs, b.reshape(1, fi), cnt, Wmu, bmu.reshape(1, fi),
      Wlv, blv.reshape(1, fi), eps, Wd1)


def _mm_final(S, hs, b, cnt, Wdm, bdm, Wds, bds, eps2):
    fi = Wdm.shape[0]

    def body(s_ref, h_ref, b_ref, c_ref, wm_ref, bm_ref, ws_ref, bs_ref,
             e_ref, o_ref):
        dinv = lax.rsqrt(c_ref[...] + 1.0)
        d = (s_ref[...] + h_ref[...]) * dinv + b_ref[...]
        lmu = jnp.dot(d, wm_ref[...],
                      preferred_element_type=jnp.float32) + bm_ref[...]
        lls = jnp.clip(jnp.dot(d, ws_ref[...],
                               preferred_element_type=jnp.float32)
                       + bs_ref[...], -10.0, 3.0)
        o_ref[...] = jnp.exp(jnp.clip(lmu + jnp.exp(lls) * e_ref[...],
                                      -20.0, 20.0))

    return pl.pallas_call(
        body, grid=(NP // BR,),
        in_specs=[_row_spec(fi), _row_spec(fi), _full_spec(1, fi),
                  _row_spec(1), _full_spec(fi, fi), _full_spec(1, fi),
                  _full_spec(fi, fi), _full_spec(1, fi), _row_spec(fi)],
        out_specs=_row_spec(fi),
        out_shape=jax.ShapeDtypeStruct((NP, fi), jnp.float32),
    )(S, hs, b.reshape(1, fi), cnt, Wdm, bdm.reshape(1, fi),
      Wds, bds.reshape(1, fi), eps2)


def _pad_rows(a):
    f = a.shape[1]
    out = jnp.zeros((NP, f), a.dtype)
    out = lax.dynamic_update_slice(out, a[:NH], (0, 0))
    out = lax.dynamic_update_slice(out, a[NH:], (NHP, 0))
    return out


def _pad_w(W, fi, fo):
    return jnp.pad(W, ((0, fi - W.shape[0]), (0, fo - W.shape[1])))


def _pad_b(b, fo):
    return jnp.pad(b, (0, fo - b.shape[0]))


def _build_edges(edge_index):
    src = edge_index[0]
    dst = edge_index[1]
    srcrow0 = src + PADH * (src >= NH).astype(jnp.int32)
    dstrow0 = dst + PADH * (dst >= NH).astype(jnp.int32)
    key = dstrow0 // ZROWS
    order = jnp.argsort(key, stable=True)
    srcrow = srcrow0[order]
    dstrow = dstrow0[order]

    cnt_w = jnp.bincount(key, length=32).astype(jnp.int32)
    P_w = ((cnt_w + CH - 1) // CH) * CH
    CS = jnp.concatenate([jnp.zeros((1,), jnp.int32),
                          jnp.cumsum(cnt_w)[:-1].astype(jnp.int32)])
    S_w = jnp.concatenate([jnp.zeros((1,), jnp.int32),
                           jnp.cumsum(P_w)[:-1].astype(jnp.int32)])

    p = jnp.arange(EP, dtype=jnp.int32)
    w_of_p = jnp.clip(jnp.searchsorted(S_w, p, side="right") - 1, 0, 31)
    off = p - S_w[w_of_p]
    valid = off < cnt_w[w_of_p]
    jidx = jnp.clip(CS[w_of_p] + off, 0, E - 1)
    src_pad = jnp.where(valid, srcrow[jidx], NH)
    dst_pad = jnp.where(valid, dstrow[jidx] - w_of_p * ZROWS, ZROWS)

    meta = jnp.stack([
        jnp.broadcast_to((S_w // CH)[:, None], (32, 16)),
        jnp.broadcast_to((P_w // CH)[:, None], (32, 16)),
    ], axis=1).astype(jnp.int32)
    return src_pad, dst_pad, meta


def kernel(x, edge_index, We1, be1, We2, be2, We3, be3, Wmu, bmu, Wlv, blv,
           Wd1, bd1, Wd2, bd2, Wd3, bd3, Wdm, bdm, Wds, bds):
    src_pad, dst_pad, meta = _build_edges(edge_index)

    zeros = jnp.zeros((AROWS * F,), jnp.float32)

    xp = _pad_rows(x)
    eps = _pad_rows(jax.random.normal(jax.random.key(42), (N, 128), jnp.float32))
    eps2 = _pad_rows(jax.random.normal(jax.random.key(43), (N, F), jnp.float32))

    f1, f2, f3 = 256, 256, 128
    We1p = _pad_w(We1, F, f1)
    We2p = _pad_w(We2, f1, f2)
    We3p = _pad_w(We3, f2, f3)
    Wd1p = _pad_w(Wd1, f3, f2)
    Wd2p = _pad_w(Wd2, f2, f1)
    Wd3p = _pad_w(Wd3, f1, F)
    be1p = _pad_b(be1, f1)
    be2p = _pad_b(be2, f2)
    bd1p = _pad_b(bd1, f2)
    bd2p = _pad_b(bd2, f1)

    cnt = _degree_counts(dst_pad, meta, zeros[:AROWS * 16])

    hs1 = _mm_pre(xp, We1p, cnt)
    S1 = _spmm(hs1, src_pad, dst_pad, meta, zeros[:AROWS * f1], f1)
    hs2 = _mm_mid(S1, hs1, be1p, cnt, We2p)
    S2 = _spmm(hs2, src_pad, dst_pad, meta, zeros[:AROWS * f2], f2)
    hs3 = _mm_mid(S2, hs2, be2p, cnt, We3p)
    S3 = _spmm(hs3, src_pad, dst_pad, meta, zeros[:AROWS * f3], f3)
    hs4 = _mm_z(S3, hs3, be3, cnt, Wmu, bmu, Wlv, blv, eps, Wd1p)
    S4 = _spmm(hs4, src_pad, dst_pad, meta, zeros[:AROWS * f2], f2)
    hs5 = _mm_mid(S4, hs4, bd1p, cnt, Wd2p)
    S5 = _spmm(hs5, src_pad, dst_pad, meta, zeros[:AROWS * f1], f1)
    hs6 = _mm_mid(S5, hs5, bd2p, cnt, Wd3p)
    S6 = _spmm(hs6, src_pad, dst_pad, meta, zeros[:AROWS * F], F)
    outp = _mm_final(S6, hs6, bd3, cnt, Wdm, bdm, Wds, bds, eps2)

    return jnp.concatenate([outp[:NH], outp[NHP:NHP + NH]], axis=0)
